# Initial kernel scaffold; baseline (speedup 1.0000x reference)
import jax
import jax.numpy as jnp
from jax import lax
from jax.experimental import pallas as pl
from jax.experimental.pallas import tpu as pltpu

_C = 15
_HW = 20000
_K = 1024
_TOPN = 1000
_NOUT = 100
_NMS_THRESH = 0.5
_PRE_THRESH = 0.05

_INTERPRET = False


def _nms_tc_body(gat_ref, val_ref, idx_ref, out_ref, lab_ref):
    gat = gat_ref[0]
    val = val_ref[0]
    idx = idx_ref[0]

    locx = gat[0]
    locy = gat[1]
    reg0 = gat[2]
    reg1 = gat[3]
    reg2 = gat[4]
    reg3 = gat[5]
    ctrx = gat[6]
    ctry = gat[7]

    cls = jnp.bitwise_and(idx, 15)
    valid = val > 0.0

    pbr_w = reg0 + reg1
    pbr_h = reg2 + reg3
    cx = locx + ctrx
    cy = locy + ctry
    x1 = cx - pbr_w / 2.0
    y1 = cy - pbr_h / 2.0
    x2 = cx + pbr_w / 2.0
    y2 = cy + pbr_h / 2.0
    w0 = reg0
    h0 = reg2
    p1x = x1 + w0
    p1y = y1
    p2x = x2
    p2y = y1 + h0
    p3x = x2 - w0
    p3y = y2
    p4x = x1
    p4y = y2 - h0
    ccx = (p1x + p2x + p3x + p4x) * 0.25
    ccy = (p1y + p2y + p3y + p4y) * 0.25
    ex_ = p2x - p1x
    ey_ = p2y - p1y
    angle = jnp.arctan2(ey_, ex_)
    ca = jnp.cos(-angle)
    sa = jnp.sin(-angle)

    def rot_xy(px, py):
        dx = px - ccx
        dy = py - ccy
        return ca * dx - sa * dy, sa * dx + ca * dy

    r1x, r1y = rot_xy(p1x, p1y)
    r2x, r2y = rot_xy(p2x, p2y)
    r3x, r3y = rot_xy(p3x, p3y)
    r4x, r4y = rot_xy(p4x, p4y)
    rw = (jnp.maximum(jnp.maximum(r1x, r2x), jnp.maximum(r3x, r4x))
          - jnp.minimum(jnp.minimum(r1x, r2x), jnp.minimum(r3x, r4x)))
    rh = (jnp.maximum(jnp.maximum(r1y, r2y), jnp.maximum(r3y, r4y))
          - jnp.minimum(jnp.minimum(r1y, r2y), jnp.minimum(r3y, r4y)))

    caa = jnp.abs(jnp.cos(angle))
    saa = jnp.abs(jnp.sin(angle))
    exh = (rw * caa + rh * saa) / 2.0
    eyh = (rw * saa + rh * caa) / 2.0
    bx1 = ccx - exh
    by1 = ccy - eyh
    bx2 = ccx + exh
    by2 = ccy + eyh
    area = jnp.maximum(bx2 - bx1, 0.0) * jnp.maximum(by2 - by1, 0.0)

    ix1 = jnp.maximum(bx1[:, None], bx1[None, :])
    iy1 = jnp.maximum(by1[:, None], by1[None, :])
    ix2 = jnp.minimum(bx2[:, None], bx2[None, :])
    iy2 = jnp.minimum(by2[:, None], by2[None, :])
    inter = jnp.maximum(ix2 - ix1, 0.0) * jnp.maximum(iy2 - iy1, 0.0)
    iou = inter / (area[:, None] + area[None, :] - inter + 1e-9)
    same = cls[:, None] == cls[None, :]
    prec = (val[:, None] > val[None, :]) | (
        (val[:, None] == val[None, :]) & (idx[:, None] < idx[None, :]))
    supf = jnp.where(
        (iou > _NMS_THRESH) & same & prec & valid[:, None], 1.0, 0.0)

    validf = jnp.where(valid, 1.0, 0.0)

    def cond(c):
        _, changed, it = c
        return changed & (it < _K)

    def body(c):
        keep, _, it = c
        hit = jnp.max(supf * keep[:, None], axis=0)
        new = validf * (1.0 - hit)
        changed = jnp.any(new != keep)
        return new, changed, it + 1

    keepf, _, _ = lax.while_loop(
        cond, body, (validf, jnp.bool_(True), jnp.int32(0)))

    kgt = (keepf[:, None] > keepf[None, :]) | (
        (keepf[:, None] == keepf[None, :]) & prec)
    rank = jnp.sum(jnp.where(kgt, 1.0, 0.0), axis=0)

    rp = lax.broadcasted_iota(jnp.float32, (_NOUT, _K), 0)
    oh = jnp.where(rank[None, :] == rp, 1.0, 0.0)

    sc = jnp.sqrt(jnp.maximum(val, 1e-12)) * validf
    score_out = keepf * sc
    lab_pay = jnp.where(keepf > 0.0, cls.astype(jnp.float32), -1.0)

    def sel(v):
        return jnp.sum(oh * v[None, :], axis=1)

    out_ref[0] = jnp.stack(
        [sel(ccx), sel(ccy), sel(rw), sel(rh), sel(angle), sel(score_out)],
        axis=1)
    lab_ref[0] = sel(lab_pay).astype(jnp.int32)


def _nms_tc(gat, val, idx):
    return pl.pallas_call(
        _nms_tc_body,
        grid=(2,),
        in_specs=[
            pl.BlockSpec((1, 8, _K), lambda i: (i, 0, 0)),
            pl.BlockSpec((1, _K), lambda i: (i, 0)),
            pl.BlockSpec((1, _K), lambda i: (i, 0)),
        ],
        out_specs=[
            pl.BlockSpec((1, _NOUT, 6), lambda i: (i, 0, 0)),
            pl.BlockSpec((1, _NOUT), lambda i: (i, 0)),
        ],
        out_shape=[
            jax.ShapeDtypeStruct((2, _NOUT, 6), jnp.float32),
            jax.ShapeDtypeStruct((2, _NOUT), jnp.int32),
        ],
        interpret=_INTERPRET,
    )(gat, val, idx)


def _select_scaffold(locations, box_cls, box_regression, center, confs):
    bc = jax.nn.sigmoid(box_cls.reshape(2, _C, _HW))
    conf = jax.nn.sigmoid(confs.reshape(2, _HW))
    bct = bc.transpose(0, 2, 1)
    cand = bct > _PRE_THRESH
    score = jnp.where(cand, bct * conf[..., None], 0.0).reshape(2, -1)
    vals, idx = lax.top_k(score, _TOPN)
    hw = idx // _C
    c = idx % _C
    idx16 = hw * 16 + c
    regt = box_regression.reshape(2, 4, _HW).transpose(0, 2, 1)
    ctrt = center.reshape(2, 2, _HW).transpose(0, 2, 1)
    gat_list = []
    for img in range(2):
        row = jnp.concatenate(
            [locations[hw[img]], regt[img][hw[img]], ctrt[img][hw[img]]],
            axis=1)
        gat_list.append(row.T)
    gat = jnp.stack(gat_list)
    pad = _K - _TOPN
    gat = jnp.pad(gat, ((0, 0), (0, 0), (0, pad)))
    vals = jnp.pad(vals, ((0, 0), (0, pad)), constant_values=-1.0)
    idx16 = jnp.pad(idx16, ((0, 0), (0, pad)))
    return gat, vals, idx16.astype(jnp.int32)


def kernel(locations, box_cls, box_regression, center, confs):
    gat, vals, idx16 = _select_scaffold(
        locations, box_cls, box_regression, center, confs)
    out, labels = _nms_tc(gat, vals, idx16)
    return out, labels


# speedup vs baseline: 1.0527x; 1.0527x over previous
import jax
import jax.numpy as jnp
from jax import lax
from jax.experimental import pallas as pl
from jax.experimental.pallas import tpu as pltpu

_C = 15
_HW = 20000
_K = 1024
_TOPN = 1000
_NOUT = 100
_NMS_THRESH = 0.5
_PRE_THRESH = 0.05

_INTERPRET = False


def _decode(locx, locy, reg0, reg1, reg2, reg3, ctrx, ctry):
    pbr_w = reg0 + reg1
    pbr_h = reg2 + reg3
    cx = locx + ctrx
    cy = locy + ctry
    x1 = cx - pbr_w / 2.0
    y1 = cy - pbr_h / 2.0
    x2 = cx + pbr_w / 2.0
    y2 = cy + pbr_h / 2.0
    w0 = reg0
    h0 = reg2
    p1x = x1 + w0
    p1y = y1
    p2x = x2
    p2y = y1 + h0
    p3x = x2 - w0
    p3y = y2
    p4x = x1
    p4y = y2 - h0
    ccx = (p1x + p2x + p3x + p4x) * 0.25
    ccy = (p1y + p2y + p3y + p4y) * 0.25
    angle = jnp.arctan2(p2y - p1y, p2x - p1x)
    ca = jnp.cos(-angle)
    sa = jnp.sin(-angle)

    def rot_xy(px, py):
        dx = px - ccx
        dy = py - ccy
        return ca * dx - sa * dy, sa * dx + ca * dy

    r1x, r1y = rot_xy(p1x, p1y)
    r2x, r2y = rot_xy(p2x, p2y)
    r3x, r3y = rot_xy(p3x, p3y)
    r4x, r4y = rot_xy(p4x, p4y)
    rw = (jnp.maximum(jnp.maximum(r1x, r2x), jnp.maximum(r3x, r4x))
          - jnp.minimum(jnp.minimum(r1x, r2x), jnp.minimum(r3x, r4x)))
    rh = (jnp.maximum(jnp.maximum(r1y, r2y), jnp.maximum(r3y, r4y))
          - jnp.minimum(jnp.minimum(r1y, r2y), jnp.minimum(r3y, r4y)))

    caa = jnp.abs(jnp.cos(angle))
    saa = jnp.abs(jnp.sin(angle))
    exh = (rw * caa + rh * saa) / 2.0
    eyh = (rw * saa + rh * caa) / 2.0
    bx1 = ccx - exh
    by1 = ccy - eyh
    bx2 = ccx + exh
    by2 = ccy + eyh
    area = jnp.maximum(bx2 - bx1, 0.0) * jnp.maximum(by2 - by1, 0.0)
    return ccx, ccy, rw, rh, angle, bx1, by1, bx2, by2, area


def _nms_tc_body(gat_ref, gatt_ref, val_ref, valt_ref, idx_ref, idxt_ref,
                 out_ref, lab_ref):
    gat = gat_ref[0]
    gatt = gatt_ref[0]
    val = val_ref[0, 0:1, :]
    valt = valt_ref[0]
    idx = idx_ref[0, 0:1, :]
    idxt = idxt_ref[0]

    rowq = [gat[q:q + 1, :] for q in range(8)]
    colq = [gatt[:, q:q + 1] for q in range(8)]

    (ccx, ccy, rw, rh, angle, bx1, by1, bx2, by2, area) = _decode(*rowq)
    (_, _, _, _, _, cbx1, cby1, cbx2, cby2, carea) = _decode(*colq)

    cls = jnp.bitwise_and(idx, 15)
    clsc = jnp.bitwise_and(idxt, 15)
    validf = jnp.where(val > 0.0, 1.0, 0.0)
    validcf = jnp.where(valt > 0.0, 1.0, 0.0)

    ix1 = jnp.maximum(cbx1, bx1)
    iy1 = jnp.maximum(cby1, by1)
    ix2 = jnp.minimum(cbx2, bx2)
    iy2 = jnp.minimum(cby2, by2)
    inter = jnp.maximum(ix2 - ix1, 0.0) * jnp.maximum(iy2 - iy1, 0.0)
    iou = inter / (carea + area - inter + 1e-9)

    same = clsc == cls
    prec = (valt > val) | ((valt == val) & (idxt < idx))
    supb = jnp.where(
        (iou > _NMS_THRESH) & same & prec & (validcf > 0.0), 1.0, 0.0
    ).astype(jnp.bfloat16)
    precf = jnp.where(prec, 1.0, 0.0)

    def cond(c):
        _, changed, it = c
        return changed & (it < _K)

    def body(c):
        keep, _, it = c
        hit = jax.lax.dot_general(
            keep.astype(jnp.bfloat16), supb,
            (((1,), (0,)), ((), ())),
            preferred_element_type=jnp.float32)
        new = validf * jnp.where(hit > 0.0, 0.0, 1.0)
        changed = jnp.any(new != keep)
        return new, changed, it + 1

    keepf, _, _ = lax.while_loop(
        cond, body, (validf, jnp.bool_(True), jnp.int32(0)))

    a_row = jax.lax.dot_general(
        keepf.astype(jnp.bfloat16), precf.astype(jnp.bfloat16),
        (((1,), (0,)), ((), ())),
        preferred_element_type=jnp.float32)
    p_row = jnp.sum(precf, axis=0, keepdims=True)
    nkeep = jnp.sum(keepf)
    rank = jnp.where(keepf > 0.0, a_row, nkeep + p_row - a_row)

    rp = lax.broadcasted_iota(jnp.int32, (_NOUT, _K), 0).astype(jnp.float32)
    oh = jnp.where(rank == rp, 1.0, 0.0)

    sc = jnp.sqrt(jnp.maximum(val, 1e-12)) * validf
    score_out = keepf * sc
    lab_pay = jnp.where(keepf > 0.0, cls.astype(jnp.float32), -1.0)

    def sel(v):
        return jnp.sum(oh * v, axis=1, keepdims=True)

    out_ref[0] = jnp.concatenate(
        [sel(ccx), sel(ccy), sel(rw), sel(rh), sel(angle), sel(score_out)],
        axis=1)
    lab_ref[0] = sel(lab_pay).astype(jnp.int32)


def _nms_tc(gat, gatt, val, valt, idx, idxt):
    return pl.pallas_call(
        _nms_tc_body,
        grid=(2,),
        in_specs=[
            pl.BlockSpec((1, 8, _K), lambda i: (i, 0, 0)),
            pl.BlockSpec((1, _K, 8), lambda i: (i, 0, 0)),
            pl.BlockSpec((1, 1, _K), lambda i: (i, 0, 0)),
            pl.BlockSpec((1, _K, 1), lambda i: (i, 0, 0)),
            pl.BlockSpec((1, 1, _K), lambda i: (i, 0, 0)),
            pl.BlockSpec((1, _K, 1), lambda i: (i, 0, 0)),
        ],
        out_specs=[
            pl.BlockSpec((1, _NOUT, 6), lambda i: (i, 0, 0)),
            pl.BlockSpec((1, _NOUT, 1), lambda i: (i, 0, 0)),
        ],
        out_shape=[
            jax.ShapeDtypeStruct((2, _NOUT, 6), jnp.float32),
            jax.ShapeDtypeStruct((2, _NOUT, 1), jnp.int32),
        ],
        interpret=_INTERPRET,
    )(gat, gatt, val, valt, idx, idxt)


def _select_scaffold(locations, box_cls, box_regression, center, confs):
    bc = jax.nn.sigmoid(box_cls.reshape(2, _C, _HW))
    conf = jax.nn.sigmoid(confs.reshape(2, _HW))
    bct = bc.transpose(0, 2, 1)
    cand = bct > _PRE_THRESH
    score = jnp.where(cand, bct * conf[..., None], 0.0).reshape(2, -1)
    vals, idx = lax.top_k(score, _TOPN)
    hw = idx // _C
    c = idx % _C
    idx16 = hw * 16 + c
    regt = box_regression.reshape(2, 4, _HW).transpose(0, 2, 1)
    ctrt = center.reshape(2, 2, _HW).transpose(0, 2, 1)
    gat_list = []
    for img in range(2):
        row = jnp.concatenate(
            [locations[hw[img]], regt[img][hw[img]], ctrt[img][hw[img]]],
            axis=1)
        gat_list.append(row.T)
    gat = jnp.stack(gat_list)
    pad = _K - _TOPN
    gat = jnp.pad(gat, ((0, 0), (0, 0), (0, pad)))
    vals = jnp.pad(vals, ((0, 0), (0, pad)), constant_values=-1.0)
    idx16 = jnp.pad(idx16, ((0, 0), (0, pad)))
    return gat, vals, idx16.astype(jnp.int32)


def kernel(locations, box_cls, box_regression, center, confs):
    gat, vals, idx16 = _select_scaffold(
        locations, box_cls, box_regression, center, confs)
    gatt = gat.transpose(0, 2, 1)
    val3 = vals.reshape(2, 1, _K)
    valt = vals.reshape(2, _K, 1)
    idx3 = idx16.reshape(2, 1, _K)
    idxt = idx16.reshape(2, _K, 1)
    out, labels = _nms_tc(gat, gatt, val3, valt, idx3, idxt)
    return out, labels.reshape(2, _NOUT)


# speedup vs baseline: 11.9430x; 11.3456x over previous
import functools

import jax
import jax.numpy as jnp
from jax import lax
from jax.experimental import pallas as pl
from jax.experimental.pallas import tpu as pltpu
from jax.experimental.pallas import tpu_sc as plsc

_C = 15
_HW = 20000
_K = 1024
_TOPN = 1000
_NOUT = 100
_NMS_THRESH = 0.5
_PRE_THRESH = 0.05

_INTERPRET = False


def _decode(locx, locy, reg0, reg1, reg2, reg3, ctrx, ctry):
    pbr_w = reg0 + reg1
    pbr_h = reg2 + reg3
    cx = locx + ctrx
    cy = locy + ctry
    x1 = cx - pbr_w / 2.0
    y1 = cy - pbr_h / 2.0
    x2 = cx + pbr_w / 2.0
    y2 = cy + pbr_h / 2.0
    w0 = reg0
    h0 = reg2
    p1x = x1 + w0
    p1y = y1
    p2x = x2
    p2y = y1 + h0
    p3x = x2 - w0
    p3y = y2
    p4x = x1
    p4y = y2 - h0
    ccx = (p1x + p2x + p3x + p4x) * 0.25
    ccy = (p1y + p2y + p3y + p4y) * 0.25
    angle = jnp.arctan2(p2y - p1y, p2x - p1x)
    ca = jnp.cos(-angle)
    sa = jnp.sin(-angle)

    def rot_xy(px, py):
        dx = px - ccx
        dy = py - ccy
        return ca * dx - sa * dy, sa * dx + ca * dy

    r1x, r1y = rot_xy(p1x, p1y)
    r2x, r2y = rot_xy(p2x, p2y)
    r3x, r3y = rot_xy(p3x, p3y)
    r4x, r4y = rot_xy(p4x, p4y)
    rw = (jnp.maximum(jnp.maximum(r1x, r2x), jnp.maximum(r3x, r4x))
          - jnp.minimum(jnp.minimum(r1x, r2x), jnp.minimum(r3x, r4x)))
    rh = (jnp.maximum(jnp.maximum(r1y, r2y), jnp.maximum(r3y, r4y))
          - jnp.minimum(jnp.minimum(r1y, r2y), jnp.minimum(r3y, r4y)))

    caa = jnp.abs(jnp.cos(angle))
    saa = jnp.abs(jnp.sin(angle))
    exh = (rw * caa + rh * saa) / 2.0
    eyh = (rw * saa + rh * caa) / 2.0
    bx1 = ccx - exh
    by1 = ccy - eyh
    bx2 = ccx + exh
    by2 = ccy + eyh
    area = jnp.maximum(bx2 - bx1, 0.0) * jnp.maximum(by2 - by1, 0.0)
    return ccx, ccy, rw, rh, angle, bx1, by1, bx2, by2, area


def _nms_tc_body(gat_ref, gatt_ref, val_ref, valt_ref, idx_ref, idxt_ref,
                 out_ref, lab_ref):
    gat = gat_ref[0]
    gatt = gatt_ref[0]
    val = val_ref[0, 0:1, :]
    valt = valt_ref[0]
    idx = idx_ref[0, 0:1, :]
    idxt = idxt_ref[0]

    rowq = [gat[q:q + 1, :] for q in range(8)]
    colq = [gatt[:, q:q + 1] for q in range(8)]

    (ccx, ccy, rw, rh, angle, bx1, by1, bx2, by2, area) = _decode(*rowq)
    (_, _, _, _, _, cbx1, cby1, cbx2, cby2, carea) = _decode(*colq)

    cls = jnp.bitwise_and(idx, 15)
    clsc = jnp.bitwise_and(idxt, 15)
    validf = jnp.where(val > 0.0, 1.0, 0.0)
    validcf = jnp.where(valt > 0.0, 1.0, 0.0)

    ix1 = jnp.maximum(cbx1, bx1)
    iy1 = jnp.maximum(cby1, by1)
    ix2 = jnp.minimum(cbx2, bx2)
    iy2 = jnp.minimum(cby2, by2)
    inter = jnp.maximum(ix2 - ix1, 0.0) * jnp.maximum(iy2 - iy1, 0.0)
    iou = inter / (carea + area - inter + 1e-9)

    same = clsc == cls
    prec = (valt > val) | ((valt == val) & (idxt < idx))
    supb = jnp.where(
        (iou > _NMS_THRESH) & same & prec & (validcf > 0.0), 1.0, 0.0
    ).astype(jnp.bfloat16)
    precf = jnp.where(prec, 1.0, 0.0)

    def cond(c):
        _, changed, it = c
        return changed & (it < _K)

    def body(c):
        keep, _, it = c
        hit = jax.lax.dot_general(
            keep.astype(jnp.bfloat16), supb,
            (((1,), (0,)), ((), ())),
            preferred_element_type=jnp.float32)
        new = validf * jnp.where(hit > 0.0, 0.0, 1.0)
        changed = jnp.any(new != keep)
        return new, changed, it + 1

    keepf, _, _ = lax.while_loop(
        cond, body, (validf, jnp.bool_(True), jnp.int32(0)))

    a_row = jax.lax.dot_general(
        keepf.astype(jnp.bfloat16), precf.astype(jnp.bfloat16),
        (((1,), (0,)), ((), ())),
        preferred_element_type=jnp.float32)
    p_row = jnp.sum(precf, axis=0, keepdims=True)
    nkeep = jnp.sum(keepf)
    rank = jnp.where(keepf > 0.0, a_row, nkeep + p_row - a_row)

    rp = lax.broadcasted_iota(jnp.int32, (_NOUT, _K), 0).astype(jnp.float32)
    oh = jnp.where(rank == rp, 1.0, 0.0)

    sc = jnp.sqrt(jnp.maximum(val, 1e-12)) * validf
    score_out = keepf * sc
    lab_pay = jnp.where(keepf > 0.0, cls.astype(jnp.float32), -1.0)

    def sel(v):
        return jnp.sum(oh * v, axis=1, keepdims=True)

    out_ref[0] = jnp.concatenate(
        [sel(ccx), sel(ccy), sel(rw), sel(rh), sel(angle), sel(score_out)],
        axis=1)
    lab_ref[0] = sel(lab_pay).astype(jnp.int32)


def _nms_tc(gat, gatt, val, valt, idx, idxt):
    return pl.pallas_call(
        _nms_tc_body,
        grid=(2,),
        in_specs=[
            pl.BlockSpec((1, 8, _K), lambda i: (i, 0, 0)),
            pl.BlockSpec((1, _K, 8), lambda i: (i, 0, 0)),
            pl.BlockSpec((1, 1, _K), lambda i: (i, 0, 0)),
            pl.BlockSpec((1, _K, 1), lambda i: (i, 0, 0)),
            pl.BlockSpec((1, 1, _K), lambda i: (i, 0, 0)),
            pl.BlockSpec((1, _K, 1), lambda i: (i, 0, 0)),
        ],
        out_specs=[
            pl.BlockSpec((1, _NOUT, 6), lambda i: (i, 0, 0)),
            pl.BlockSpec((1, _NOUT, 1), lambda i: (i, 0, 0)),
        ],
        out_shape=[
            jax.ShapeDtypeStruct((2, _NOUT, 6), jnp.float32),
            jax.ShapeDtypeStruct((2, _NOUT, 1), jnp.int32),
        ],
        interpret=_INTERPRET,
    )(gat, gatt, val, valt, idx, idxt)


_PASSES = ((23, 31, 8), (15, 23, 8), (7, 15, 8), (0, 7, 7))
_ROWS = _HW // 16
_ROWS_PAD = 1264
_NSEL = 1056


def _sc_select(clsp, conf2, tbl):
    mesh = plsc.VectorSubcoreMesh(core_axis_name="c", subcore_axis_name="s",
                                  num_cores=2, num_subcores=16)

    @functools.partial(
        pl.kernel,
        out_type=[
            jax.ShapeDtypeStruct((2 * _K,), jnp.float32),
            jax.ShapeDtypeStruct((2 * _K,), jnp.int32),
            jax.ShapeDtypeStruct((2 * 8 * _K,), jnp.float32),
        ],
        mesh=mesh,
        compiler_params=pltpu.CompilerParams(needs_layout_passes=False),
        scratch_types=[
            pltpu.VMEM((_ROWS * 16,), jnp.float32),
            pltpu.VMEM((_ROWS_PAD,), jnp.float32),
            pltpu.VMEM((_ROWS * 16,), jnp.int32),
            [pltpu.VMEM((256,), jnp.int32)] * 4,
            pltpu.VMEM((16, 256), jnp.int32),
            pltpu.VMEM((512,), jnp.int32),
            [pltpu.VMEM((16,), jnp.int32)] * 2,
            pltpu.VMEM((1024,), jnp.float32),
            pltpu.VMEM((1024,), jnp.int32),
            pltpu.VMEM((8, 128), jnp.int32),
            pltpu.VMEM((64,), jnp.int32),
            pltpu.VMEM((64,), jnp.float32),
            pltpu.VMEM((64,), jnp.int32),
            pltpu.VMEM((64,), jnp.float32),
            pltpu.VMEM((16,), jnp.float32),
            pltpu.VMEM((16,), jnp.int32),
            pltpu.VMEM_SHARED((16, 256), jnp.int32),
            pltpu.VMEM_SHARED((512,), jnp.int32),
            pltpu.VMEM_SHARED((_NSEL,), jnp.float32),
            pltpu.VMEM_SHARED((_NSEL,), jnp.int32),
            pltpu.SemaphoreType.DMA,
        ],
    )
    def sel_kernel(clsp_hbm, conf_hbm, tbl_hbm, val_hbm, idx_hbm, gatt_hbm,
                   cls_v, conf_v, bits_v, lhist_v, ghist_v, cnts_v, myc_v,
                   selv_v, seli_v, pos_v, didx_v, dval_v, dhw_v, rows_v,
                   padf_v, padi_v, shist, scnt, sval_sp, sidx_sp, sem):
        cid = lax.axis_index("c")
        sid = lax.axis_index("s")
        lane = lax.broadcasted_iota(jnp.int32, (16,), 0)
        lane_ok = lane < _C
        zeros16 = jnp.zeros((16,), jnp.int32)

        wid = cid * 16 + sid
        pltpu.sync_copy(clsp_hbm.at[pl.ds(wid * (_ROWS * 16), _ROWS * 16)],
                        cls_v)
        pltpu.sync_copy(conf_hbm.at[pl.ds(wid * _ROWS_PAD, _ROWS_PAD)],
                        conf_v)

        def conf_chunk(j, _):
            x = conf_v[pl.ds(j * 16, 16)]
            conf_v[pl.ds(j * 16, 16)] = 1.0 / (1.0 + jnp.exp(-x))
            return 0

        lax.fori_loop(0, _ROWS_PAD // 16, conf_chunk, 0)

        def score_rows(r16, i_range, cf):
            for i in range(i_range):
                r = r16 * 16 + i
                row = cls_v[pl.ds(r * 16, 16)]
                s = 1.0 / (1.0 + jnp.exp(-row))
                sc = jnp.where((s > _PRE_THRESH) & lane_ok, s * cf[i], 0.0)
                bits_v[pl.ds(r * 16, 16)] = lax.bitcast_convert_type(
                    sc, jnp.int32)

        def score_chunk(r16, _):
            score_rows(r16, 16, conf_v[pl.ds(r16 * 16, 16)])
            return 0

        lax.fori_loop(0, _ROWS // 16, score_chunk, 0)
        score_rows(_ROWS // 16, _ROWS % 16,
                   conf_v[pl.ds((_ROWS // 16) * 16, 16)])

        prefix = jnp.int32(0)
        r_rank = jnp.int32(_TOPN)
        digits = []
        for p, (sh, sh2, w) in enumerate(_PASSES):
            nbins = 1 << w
            hist_p = lhist_v[p]
            for j in range(nbins // 16):
                hist_p[pl.ds(j * 16, 16)] = zeros16

            def hist_row(r, _, sh=sh, sh2=sh2, nbins=nbins, hist_p=hist_p,
                         prefix=prefix):
                b = bits_v[pl.ds(r * 16, 16)]
                hi = lax.shift_right_logical(b, sh2)
                m = (hi == prefix) & lane_ok
                d = jnp.bitwise_and(lax.shift_right_logical(b, sh), nbins - 1)
                cnt, lastm = plsc.scan_count(d, m)
                plsc.addupdate_scatter(hist_p, [d], cnt, mask=lastm & m)
                return 0

            lax.fori_loop(0, _ROWS, hist_row, 0, unroll=2)

            pltpu.sync_copy(hist_p, shist.at[sid])
            plsc.subcore_barrier()
            pltpu.sync_copy(shist, ghist_v)
            plsc.subcore_barrier()
            carry = jnp.int32(0)
            dacc = jnp.int32(0)
            sufd = jnp.int32(0)
            for j in reversed(range(nbins // 16)):
                acc = zeros16
                for t in range(16):
                    acc = acc + ghist_v[t, pl.ds(j * 16, 16)]
                csum = plsc.cumsum(acc)
                tot = jnp.sum(acc)
                suffix = carry + (tot - csum)
                m = (suffix < r_rank) & (suffix + acc >= r_rank)
                binidx = j * 16 + lane
                dacc = dacc + jnp.sum(jnp.where(m, binidx, 0))
                sufd = sufd + jnp.sum(jnp.where(m, suffix, 0))
                carry = carry + tot
            digits.append(dacc)
            r_rank = r_rank - sufd
            prefix = jnp.bitwise_or(lax.shift_left(prefix, w), dacc)

        tkey = prefix
        fill = r_rank

        n_gt = jnp.int32(0)
        for p, (sh, sh2, w) in enumerate(_PASSES):
            nbins = 1 << w
            for j in range(nbins // 16):
                binidx = j * 16 + lane
                h = lhist_v[p][pl.ds(j * 16, 16)]
                n_gt = n_gt + jnp.sum(jnp.where(binidx > digits[p], h, 0))
        n_eq = jnp.int32(0)
        for j in range(8):
            binidx = j * 16 + lane
            h = lhist_v[3][pl.ds(j * 16, 16)]
            n_eq = n_eq + jnp.sum(jnp.where(binidx == digits[3], h, 0))

        myc_v[0][...] = jnp.where(lane == sid, n_gt, 0)
        myc_v[1][...] = jnp.where(lane == sid, n_eq, 0)
        pltpu.sync_copy(myc_v[0], scnt.at[pl.ds(sid * 16, 16)])
        pltpu.sync_copy(myc_v[1], scnt.at[pl.ds((16 + sid) * 16, 16)])
        plsc.subcore_barrier()
        pltpu.sync_copy(scnt, cnts_v)
        plsc.subcore_barrier()
        gt_v = zeros16
        eq_v = zeros16
        for t in range(16):
            gt_v = gt_v + cnts_v[pl.ds(t * 16, 16)]
            eq_v = eq_v + cnts_v[pl.ds((16 + t) * 16, 16)]
        gt_incl = plsc.cumsum(gt_v)
        eq_incl = plsc.cumsum(eq_v)
        total_gt = jnp.sum(gt_v)
        gt_off = jnp.sum(jnp.where(lane == sid, gt_incl - gt_v, 0))
        eq_off = jnp.sum(jnp.where(lane == sid, eq_incl - eq_v, 0))
        quota = jnp.clip(fill - eq_off, 0, n_eq)
        eq_base = total_gt + jnp.minimum(eq_off, fill)

        trash = 1024 + sid * 2 + jnp.bitwise_and(lane, 1)
        for j in range(8):
            for i in range(8):
                pos_v[j, pl.ds(i * 16, 16)] = trash

        def emit_row(rr, carry):
            out_n, gtk, eqk = carry
            b = bits_v[pl.ds(rr * 16, 16)]
            m_gt = (b > tkey) & lane_ok
            m_eq = (b == tkey) & lane_ok
            cg = plsc.cumsum(jnp.where(m_gt, 1, 0))
            ce = plsc.cumsum(jnp.where(m_eq, 1, 0))
            take_eq = m_eq & (eqk + ce <= quota)
            m_take = m_gt | take_eq
            pos = jnp.where(m_gt, gt_off + gtk + cg - 1,
                            eq_base + eqk + ce - 1)
            ct = plsc.cumsum(jnp.where(m_take, 1, 0))
            lpos = out_n + ct - 1
            fidx = (sid * _ROWS + rr) * 16 + lane
            valrow = lax.bitcast_convert_type(b, jnp.float32)
            plsc.store_scatter(selv_v, [lpos], valrow, mask=m_take)
            plsc.store_scatter(seli_v, [lpos], fidx, mask=m_take)
            plsc.store_scatter(
                pos_v,
                [lax.shift_right_logical(lpos, 7),
                 jnp.bitwise_and(lpos, 127)],
                pos, mask=m_take)
            return (out_n + jnp.sum(jnp.where(m_take, 1, 0)),
                    gtk + jnp.sum(jnp.where(m_gt, 1, 0)),
                    eqk + jnp.sum(jnp.where(m_eq, 1, 0)))

        out_n, _, _ = lax.fori_loop(
            0, _ROWS, emit_row,
            (jnp.int32(0), jnp.int32(0), jnp.int32(0)), unroll=2)

        for j in range(8):
            @pl.when(j * 128 < out_n)
            def _(j=j):
                pltpu.sync_copy(selv_v.at[pl.ds(j * 128, 128)],
                                sval_sp.at[pos_v.at[j]])
                pltpu.sync_copy(seli_v.at[pl.ds(j * 128, 128)],
                                sidx_sp.at[pos_v.at[j]])

        @pl.when(sid == 0)
        def _():
            padf_v[...] = jnp.full((16,), -1.0, jnp.float32)
            padi_v[...] = jnp.zeros((16,), jnp.int32)
            for j in range(2):
                pltpu.sync_copy(
                    padf_v, sval_sp.at[pl.ds(1000 + 16 * j, 16)])
                pltpu.sync_copy(
                    padi_v, sidx_sp.at[pl.ds(1000 + 16 * j, 16)])

        plsc.subcore_barrier()

        base = sid * 64
        obase = cid * _K + base
        pltpu.sync_copy(sidx_sp.at[pl.ds(base, 64)], didx_v)
        pltpu.sync_copy(sval_sp.at[pl.ds(base, 64)], dval_v)
        pltpu.sync_copy(didx_v, idx_hbm.at[pl.ds(obase, 64)])
        pltpu.sync_copy(dval_v, val_hbm.at[pl.ds(obase, 64)])
        tbase = cid * (_HW * 8)
        for q in range(8):
            for i in range(4):
                hw = lax.shift_right_logical(didx_v[pl.ds(i * 16, 16)], 4)
                dhw_v[pl.ds(i * 16, 16)] = tbase + hw * 8 + q
            pltpu.async_copy(tbl_hbm.at[dhw_v], rows_v, sem).wait()
            pltpu.sync_copy(
                rows_v,
                gatt_hbm.at[pl.ds((cid * 8 + q) * _K + base, 64)])

    return sel_kernel(clsp, conf2, tbl)


def _select_scaffold(locations, box_cls, box_regression, center, confs):
    bc = jax.nn.sigmoid(box_cls.reshape(2, _C, _HW))
    conf = jax.nn.sigmoid(confs.reshape(2, _HW))
    bct = bc.transpose(0, 2, 1)
    cand = bct > _PRE_THRESH
    score = jnp.where(cand, bct * conf[..., None], 0.0).reshape(2, -1)
    vals, idx = lax.top_k(score, _TOPN)
    hw = idx // _C
    c = idx % _C
    idx16 = hw * 16 + c
    regt = box_regression.reshape(2, 4, _HW).transpose(0, 2, 1)
    ctrt = center.reshape(2, 2, _HW).transpose(0, 2, 1)
    gat_list = []
    for img in range(2):
        row = jnp.concatenate(
            [locations[hw[img]], regt[img][hw[img]], ctrt[img][hw[img]]],
            axis=1)
        gat_list.append(row.T)
    gat = jnp.stack(gat_list)
    pad = _K - _TOPN
    gat = jnp.pad(gat, ((0, 0), (0, 0), (0, pad)))
    vals = jnp.pad(vals, ((0, 0), (0, pad)), constant_values=-1.0)
    idx16 = jnp.pad(idx16, ((0, 0), (0, pad)))
    return gat, vals, idx16.astype(jnp.int32)


def kernel(locations, box_cls, box_regression, center, confs):
    bc = box_cls.reshape(2, _C, _HW)
    clsp = jnp.pad(bc.transpose(0, 2, 1), ((0, 0), (0, 0), (0, 1)))
    clsp = clsp.reshape(-1)
    conf2 = jnp.pad(confs.reshape(2, 16, _ROWS),
                    ((0, 0), (0, 0), (0, _ROWS_PAD - _ROWS))).reshape(-1)
    regt = box_regression.reshape(2, 4, _HW).transpose(0, 2, 1)
    ctrt = center.reshape(2, 2, _HW).transpose(0, 2, 1)
    loc2 = jnp.broadcast_to(locations[None], (2, _HW, 2))
    tbl = jnp.concatenate([loc2, regt, ctrt], axis=2).reshape(-1)

    vals, idx16, gatf = _sc_select(clsp, conf2, tbl)
    vals = vals.reshape(2, _K)
    idx16 = idx16.reshape(2, _K)
    gat = gatf.reshape(2, 8, _K)
    gatt = gat.transpose(0, 2, 1)
    val3 = vals.reshape(2, 1, _K)
    valt = vals.reshape(2, _K, 1)
    idx3 = idx16.reshape(2, 1, _K)
    idxt = idx16.reshape(2, _K, 1)
    out, labels = _nms_tc(gat, gatt, val3, valt, idx3, idxt)
    return out, labels.reshape(2, _NOUT)


# speedup vs baseline: 11.9491x; 1.0005x over previous
import functools

import jax
import jax.numpy as jnp
from jax import lax
from jax.experimental import pallas as pl
from jax.experimental.pallas import tpu as pltpu
from jax.experimental.pallas import tpu_sc as plsc

_C = 15
_HW = 20000
_K = 1024
_TOPN = 1000
_NOUT = 100
_NMS_THRESH = 0.5
_PRE_THRESH = 0.05


def _decode(locx, locy, reg0, reg1, reg2, reg3, ctrx, ctry):
    pbr_w = reg0 + reg1
    pbr_h = reg2 + reg3
    cx = locx + ctrx
    cy = locy + ctry
    x1 = cx - pbr_w / 2.0
    y1 = cy - pbr_h / 2.0
    x2 = cx + pbr_w / 2.0
    y2 = cy + pbr_h / 2.0
    w0 = reg0
    h0 = reg2
    p1x = x1 + w0
    p1y = y1
    p2x = x2
    p2y = y1 + h0
    p3x = x2 - w0
    p3y = y2
    p4x = x1
    p4y = y2 - h0
    ccx = (p1x + p2x + p3x + p4x) * 0.25
    ccy = (p1y + p2y + p3y + p4y) * 0.25
    angle = jnp.arctan2(p2y - p1y, p2x - p1x)
    ca = jnp.cos(-angle)
    sa = jnp.sin(-angle)

    def rot_xy(px, py):
        dx = px - ccx
        dy = py - ccy
        return ca * dx - sa * dy, sa * dx + ca * dy

    r1x, r1y = rot_xy(p1x, p1y)
    r2x, r2y = rot_xy(p2x, p2y)
    r3x, r3y = rot_xy(p3x, p3y)
    r4x, r4y = rot_xy(p4x, p4y)
    rw = (jnp.maximum(jnp.maximum(r1x, r2x), jnp.maximum(r3x, r4x))
          - jnp.minimum(jnp.minimum(r1x, r2x), jnp.minimum(r3x, r4x)))
    rh = (jnp.maximum(jnp.maximum(r1y, r2y), jnp.maximum(r3y, r4y))
          - jnp.minimum(jnp.minimum(r1y, r2y), jnp.minimum(r3y, r4y)))

    caa = jnp.abs(jnp.cos(angle))
    saa = jnp.abs(jnp.sin(angle))
    exh = (rw * caa + rh * saa) / 2.0
    eyh = (rw * saa + rh * caa) / 2.0
    bx1 = ccx - exh
    by1 = ccy - eyh
    bx2 = ccx + exh
    by2 = ccy + eyh
    area = jnp.maximum(bx2 - bx1, 0.0) * jnp.maximum(by2 - by1, 0.0)
    return ccx, ccy, rw, rh, angle, bx1, by1, bx2, by2, area


def _nms_tc_body(gat_ref, gatt_ref, val_ref, valt_ref, idx_ref, idxt_ref,
                 out_ref, lab_ref):
    gat = gat_ref[0]
    gatt = gatt_ref[0]
    val = val_ref[0, 0:1, :]
    valt = valt_ref[0]
    idx = idx_ref[0, 0:1, :]
    idxt = idxt_ref[0]

    rowq = [gat[q:q + 1, :] for q in range(8)]
    colq = [gatt[:, q:q + 1] for q in range(8)]

    (ccx, ccy, rw, rh, angle, bx1, by1, bx2, by2, area) = _decode(*rowq)
    (_, _, _, _, _, cbx1, cby1, cbx2, cby2, carea) = _decode(*colq)

    cls = jnp.bitwise_and(idx, 15)
    clsc = jnp.bitwise_and(idxt, 15)
    validf = jnp.where(val > 0.0, 1.0, 0.0)
    validcf = jnp.where(valt > 0.0, 1.0, 0.0)

    ix1 = jnp.maximum(cbx1, bx1)
    iy1 = jnp.maximum(cby1, by1)
    ix2 = jnp.minimum(cbx2, bx2)
    iy2 = jnp.minimum(cby2, by2)
    inter = jnp.maximum(ix2 - ix1, 0.0) * jnp.maximum(iy2 - iy1, 0.0)
    iou = inter / (carea + area - inter + 1e-9)

    same = clsc == cls
    prec = (valt > val) | ((valt == val) & (idxt < idx))
    supb = jnp.where(
        (iou > _NMS_THRESH) & same & prec & (validcf > 0.0), 1.0, 0.0
    ).astype(jnp.bfloat16)
    precf = jnp.where(prec, 1.0, 0.0)

    def cond(c):
        _, changed, it = c
        return changed & (it < _K)

    def body(c):
        keep, _, it = c
        hit = jax.lax.dot_general(
            keep.astype(jnp.bfloat16), supb,
            (((1,), (0,)), ((), ())),
            preferred_element_type=jnp.float32)
        new = validf * jnp.where(hit > 0.0, 0.0, 1.0)
        changed = jnp.any(new != keep)
        return new, changed, it + 1

    keepf, _, _ = lax.while_loop(
        cond, body, (validf, jnp.bool_(True), jnp.int32(0)))

    a_row = jax.lax.dot_general(
        keepf.astype(jnp.bfloat16), precf.astype(jnp.bfloat16),
        (((1,), (0,)), ((), ())),
        preferred_element_type=jnp.float32)
    p_row = jnp.sum(precf, axis=0, keepdims=True)
    nkeep = jnp.sum(keepf)
    rank = jnp.where(keepf > 0.0, a_row, nkeep + p_row - a_row)

    rp = lax.broadcasted_iota(jnp.int32, (_NOUT, _K), 0).astype(jnp.float32)
    oh = jnp.where(rank == rp, 1.0, 0.0)

    sc = jnp.sqrt(jnp.maximum(val, 1e-12)) * validf
    score_out = keepf * sc
    lab_pay = jnp.where(keepf > 0.0, cls.astype(jnp.float32), -1.0)

    def sel(v):
        return jnp.sum(oh * v, axis=1, keepdims=True)

    out_ref[0] = jnp.concatenate(
        [sel(ccx), sel(ccy), sel(rw), sel(rh), sel(angle), sel(score_out)],
        axis=1)
    lab_ref[0] = sel(lab_pay).astype(jnp.int32)


def _nms_tc(gat, gatt, val, valt, idx, idxt):
    return pl.pallas_call(
        _nms_tc_body,
        grid=(2,),
        in_specs=[
            pl.BlockSpec((1, 8, _K), lambda i: (i, 0, 0)),
            pl.BlockSpec((1, _K, 8), lambda i: (i, 0, 0)),
            pl.BlockSpec((1, 1, _K), lambda i: (i, 0, 0)),
            pl.BlockSpec((1, _K, 1), lambda i: (i, 0, 0)),
            pl.BlockSpec((1, 1, _K), lambda i: (i, 0, 0)),
            pl.BlockSpec((1, _K, 1), lambda i: (i, 0, 0)),
        ],
        out_specs=[
            pl.BlockSpec((1, _NOUT, 6), lambda i: (i, 0, 0)),
            pl.BlockSpec((1, _NOUT, 1), lambda i: (i, 0, 0)),
        ],
        out_shape=[
            jax.ShapeDtypeStruct((2, _NOUT, 6), jnp.float32),
            jax.ShapeDtypeStruct((2, _NOUT, 1), jnp.int32),
        ],
    )(gat, gatt, val, valt, idx, idxt)


_PASSES = ((23, 31, 8), (15, 23, 8), (7, 15, 8), (0, 7, 7))
_ROWS = _HW // 16
_ROWS_PAD = 1264
_NSEL = 1056


def _sc_select(clsp, conf2, tbl):
    mesh = plsc.VectorSubcoreMesh(core_axis_name="c", subcore_axis_name="s",
                                  num_cores=2, num_subcores=16)

    @functools.partial(
        pl.kernel,
        out_type=[
            jax.ShapeDtypeStruct((2 * _K,), jnp.float32),
            jax.ShapeDtypeStruct((2 * _K,), jnp.int32),
            jax.ShapeDtypeStruct((2 * 8 * _K,), jnp.float32),
        ],
        mesh=mesh,
        compiler_params=pltpu.CompilerParams(needs_layout_passes=False),
        scratch_types=[
            pltpu.VMEM((_ROWS * 16,), jnp.float32),
            pltpu.VMEM((_ROWS_PAD,), jnp.float32),
            pltpu.VMEM((_ROWS * 16,), jnp.int32),
            [pltpu.VMEM((256,), jnp.int32)] * 4,
            pltpu.VMEM((16, 256), jnp.int32),
            pltpu.VMEM((512,), jnp.int32),
            [pltpu.VMEM((16,), jnp.int32)] * 2,
            pltpu.VMEM((1024,), jnp.float32),
            pltpu.VMEM((1024,), jnp.int32),
            pltpu.VMEM((8, 128), jnp.int32),
            pltpu.VMEM((64,), jnp.int32),
            pltpu.VMEM((64,), jnp.float32),
            pltpu.VMEM((64,), jnp.int32),
            pltpu.VMEM((64,), jnp.float32),
            pltpu.VMEM((16,), jnp.float32),
            pltpu.VMEM((16,), jnp.int32),
            pltpu.VMEM_SHARED((16, 256), jnp.int32),
            pltpu.VMEM_SHARED((512,), jnp.int32),
            pltpu.VMEM_SHARED((_NSEL,), jnp.float32),
            pltpu.VMEM_SHARED((_NSEL,), jnp.int32),
            pltpu.SemaphoreType.DMA,
        ],
    )
    def sel_kernel(clsp_hbm, conf_hbm, tbl_hbm, val_hbm, idx_hbm, gatt_hbm,
                   cls_v, conf_v, bits_v, lhist_v, ghist_v, cnts_v, myc_v,
                   selv_v, seli_v, pos_v, didx_v, dval_v, dhw_v, rows_v,
                   padf_v, padi_v, shist, scnt, sval_sp, sidx_sp, sem):
        cid = lax.axis_index("c")
        sid = lax.axis_index("s")
        lane = lax.broadcasted_iota(jnp.int32, (16,), 0)
        lane_ok = lane < _C
        zeros16 = jnp.zeros((16,), jnp.int32)

        wid = cid * 16 + sid
        pltpu.sync_copy(clsp_hbm.at[pl.ds(wid * (_ROWS * 16), _ROWS * 16)],
                        cls_v)
        pltpu.sync_copy(conf_hbm.at[pl.ds(wid * _ROWS_PAD, _ROWS_PAD)],
                        conf_v)

        def conf_chunk(j, _):
            x = conf_v[pl.ds(j * 16, 16)]
            conf_v[pl.ds(j * 16, 16)] = 1.0 / (1.0 + jnp.exp(-x))
            return 0

        lax.fori_loop(0, _ROWS_PAD // 16, conf_chunk, 0)

        def score_rows(r16, i_range, cf):
            for i in range(i_range):
                r = r16 * 16 + i
                row = cls_v[pl.ds(r * 16, 16)]
                s = 1.0 / (1.0 + jnp.exp(-row))
                sc = jnp.where((s > _PRE_THRESH) & lane_ok, s * cf[i], 0.0)
                bits_v[pl.ds(r * 16, 16)] = lax.bitcast_convert_type(
                    sc, jnp.int32)

        def score_chunk(r16, _):
            score_rows(r16, 16, conf_v[pl.ds(r16 * 16, 16)])
            return 0

        lax.fori_loop(0, _ROWS // 16, score_chunk, 0)
        score_rows(_ROWS // 16, _ROWS % 16,
                   conf_v[pl.ds((_ROWS // 16) * 16, 16)])

        prefix = jnp.int32(0)
        r_rank = jnp.int32(_TOPN)
        digits = []
        for p, (sh, sh2, w) in enumerate(_PASSES):
            nbins = 1 << w
            hist_p = lhist_v[p]
            for j in range(nbins // 16):
                hist_p[pl.ds(j * 16, 16)] = zeros16

            def hist_row(r, _, sh=sh, sh2=sh2, nbins=nbins, hist_p=hist_p,
                         prefix=prefix):
                b = bits_v[pl.ds(r * 16, 16)]
                hi = lax.shift_right_logical(b, sh2)
                m = (hi == prefix) & lane_ok
                d = jnp.bitwise_and(lax.shift_right_logical(b, sh), nbins - 1)
                cnt, lastm = plsc.scan_count(d, m)
                plsc.addupdate_scatter(hist_p, [d], cnt, mask=lastm & m)
                return 0

            lax.fori_loop(0, _ROWS, hist_row, 0, unroll=2)

            pltpu.sync_copy(hist_p, shist.at[sid])
            plsc.subcore_barrier()
            pltpu.sync_copy(shist, ghist_v)
            plsc.subcore_barrier()
            carry = jnp.int32(0)
            dacc = jnp.int32(0)
            sufd = jnp.int32(0)
            for j in reversed(range(nbins // 16)):
                acc = zeros16
                for t in range(16):
                    acc = acc + ghist_v[t, pl.ds(j * 16, 16)]
                csum = plsc.cumsum(acc)
                tot = jnp.sum(acc)
                suffix = carry + (tot - csum)
                m = (suffix < r_rank) & (suffix + acc >= r_rank)
                binidx = j * 16 + lane
                dacc = dacc + jnp.sum(jnp.where(m, binidx, 0))
                sufd = sufd + jnp.sum(jnp.where(m, suffix, 0))
                carry = carry + tot
            digits.append(dacc)
            r_rank = r_rank - sufd
            prefix = jnp.bitwise_or(lax.shift_left(prefix, w), dacc)

        tkey = prefix
        fill = r_rank

        n_gt = jnp.int32(0)
        for p, (sh, sh2, w) in enumerate(_PASSES):
            nbins = 1 << w
            for j in range(nbins // 16):
                binidx = j * 16 + lane
                h = lhist_v[p][pl.ds(j * 16, 16)]
                n_gt = n_gt + jnp.sum(jnp.where(binidx > digits[p], h, 0))
        n_eq = jnp.int32(0)
        for j in range(8):
            binidx = j * 16 + lane
            h = lhist_v[3][pl.ds(j * 16, 16)]
            n_eq = n_eq + jnp.sum(jnp.where(binidx == digits[3], h, 0))

        myc_v[0][...] = jnp.where(lane == sid, n_gt, 0)
        myc_v[1][...] = jnp.where(lane == sid, n_eq, 0)
        pltpu.sync_copy(myc_v[0], scnt.at[pl.ds(sid * 16, 16)])
        pltpu.sync_copy(myc_v[1], scnt.at[pl.ds((16 + sid) * 16, 16)])
        plsc.subcore_barrier()
        pltpu.sync_copy(scnt, cnts_v)
        plsc.subcore_barrier()
        gt_v = zeros16
        eq_v = zeros16
        for t in range(16):
            gt_v = gt_v + cnts_v[pl.ds(t * 16, 16)]
            eq_v = eq_v + cnts_v[pl.ds((16 + t) * 16, 16)]
        gt_incl = plsc.cumsum(gt_v)
        eq_incl = plsc.cumsum(eq_v)
        total_gt = jnp.sum(gt_v)
        gt_off = jnp.sum(jnp.where(lane == sid, gt_incl - gt_v, 0))
        eq_off = jnp.sum(jnp.where(lane == sid, eq_incl - eq_v, 0))
        quota = jnp.clip(fill - eq_off, 0, n_eq)
        eq_base = total_gt + jnp.minimum(eq_off, fill)

        trash = 1024 + sid * 2 + jnp.bitwise_and(lane, 1)
        for j in range(8):
            for i in range(8):
                pos_v[j, pl.ds(i * 16, 16)] = trash

        def emit_row(rr, carry):
            out_n, gtk, eqk = carry
            b = bits_v[pl.ds(rr * 16, 16)]
            m_gt = (b > tkey) & lane_ok
            m_eq = (b == tkey) & lane_ok
            cg = plsc.cumsum(jnp.where(m_gt, 1, 0))
            ce = plsc.cumsum(jnp.where(m_eq, 1, 0))
            take_eq = m_eq & (eqk + ce <= quota)
            m_take = m_gt | take_eq
            pos = jnp.where(m_gt, gt_off + gtk + cg - 1,
                            eq_base + eqk + ce - 1)
            ct = plsc.cumsum(jnp.where(m_take, 1, 0))
            lpos = out_n + ct - 1
            fidx = (sid * _ROWS + rr) * 16 + lane
            valrow = lax.bitcast_convert_type(b, jnp.float32)
            plsc.store_scatter(selv_v, [lpos], valrow, mask=m_take)
            plsc.store_scatter(seli_v, [lpos], fidx, mask=m_take)
            plsc.store_scatter(
                pos_v,
                [lax.shift_right_logical(lpos, 7),
                 jnp.bitwise_and(lpos, 127)],
                pos, mask=m_take)
            return (out_n + jnp.sum(jnp.where(m_take, 1, 0)),
                    gtk + jnp.sum(jnp.where(m_gt, 1, 0)),
                    eqk + jnp.sum(jnp.where(m_eq, 1, 0)))

        out_n, _, _ = lax.fori_loop(
            0, _ROWS, emit_row,
            (jnp.int32(0), jnp.int32(0), jnp.int32(0)), unroll=2)

        for j in range(8):
            @pl.when(j * 128 < out_n)
            def _(j=j):
                pltpu.sync_copy(selv_v.at[pl.ds(j * 128, 128)],
                                sval_sp.at[pos_v.at[j]])
                pltpu.sync_copy(seli_v.at[pl.ds(j * 128, 128)],
                                sidx_sp.at[pos_v.at[j]])

        @pl.when(sid == 0)
        def _():
            padf_v[...] = jnp.full((16,), -1.0, jnp.float32)
            padi_v[...] = jnp.zeros((16,), jnp.int32)
            for j in range(2):
                pltpu.sync_copy(
                    padf_v, sval_sp.at[pl.ds(1000 + 16 * j, 16)])
                pltpu.sync_copy(
                    padi_v, sidx_sp.at[pl.ds(1000 + 16 * j, 16)])

        plsc.subcore_barrier()

        base = sid * 64
        obase = cid * _K + base
        pltpu.sync_copy(sidx_sp.at[pl.ds(base, 64)], didx_v)
        pltpu.sync_copy(sval_sp.at[pl.ds(base, 64)], dval_v)
        pltpu.sync_copy(didx_v, idx_hbm.at[pl.ds(obase, 64)])
        pltpu.sync_copy(dval_v, val_hbm.at[pl.ds(obase, 64)])
        tbase = cid * (_HW * 8)
        for q in range(8):
            for i in range(4):
                hw = lax.shift_right_logical(didx_v[pl.ds(i * 16, 16)], 4)
                dhw_v[pl.ds(i * 16, 16)] = tbase + hw * 8 + q
            pltpu.async_copy(tbl_hbm.at[dhw_v], rows_v, sem).wait()
            pltpu.sync_copy(
                rows_v,
                gatt_hbm.at[pl.ds((cid * 8 + q) * _K + base, 64)])

    return sel_kernel(clsp, conf2, tbl)


def kernel(locations, box_cls, box_regression, center, confs):
    bc = box_cls.reshape(2, _C, _HW)
    clsp = jnp.pad(bc.transpose(0, 2, 1), ((0, 0), (0, 0), (0, 1)))
    clsp = clsp.reshape(-1)
    conf2 = jnp.pad(confs.reshape(2, 16, _ROWS),
                    ((0, 0), (0, 0), (0, _ROWS_PAD - _ROWS))).reshape(-1)
    regt = box_regression.reshape(2, 4, _HW).transpose(0, 2, 1)
    ctrt = center.reshape(2, 2, _HW).transpose(0, 2, 1)
    loc2 = jnp.broadcast_to(locations[None], (2, _HW, 2))
    tbl = jnp.concatenate([loc2, regt, ctrt], axis=2).reshape(-1)

    vals, idx16, gatf = _sc_select(clsp, conf2, tbl)
    vals = vals.reshape(2, _K)
    idx16 = idx16.reshape(2, _K)
    gat = gatf.reshape(2, 8, _K)
    gatt = gat.transpose(0, 2, 1)
    val3 = vals.reshape(2, 1, _K)
    valt = vals.reshape(2, _K, 1)
    idx3 = idx16.reshape(2, 1, _K)
    idxt = idx16.reshape(2, _K, 1)
    out, labels = _nms_tc(gat, gatt, val3, valt, idx3, idxt)
    return out, labels.reshape(2, _NOUT)
